# baseline (speedup 1.0000x reference)
import functools

import jax
import jax.numpy as jnp
import numpy as np
from jax.experimental import pallas as pl
from jax.experimental.pallas import tpu as pltpu
from jax.experimental.pallas import tpu_sc as plsc

N = 10000
C = 256
K = 32
NS1 = 2500
NS2 = 625


def _fps_body(nreal, ns, px_ref, py_ref, pz_ref, psm_ref, out_ref, mind_ref):
    rows = px_ref.shape[0]
    flat = (jax.lax.broadcasted_iota(jnp.int32, (rows, 128), 0) * 128
            + jax.lax.broadcasted_iota(jnp.int32, (rows, 128), 1))
    mind_ref[...] = jnp.where(flat < nreal, jnp.inf, -jnp.inf)
    out_ref[0] = 0
    px = px_ref[...]
    py = py_ref[...]
    pz = pz_ref[...]

    def body(i, last):
        px0 = psm_ref[last * 3]
        py0 = psm_ref[last * 3 + 1]
        pz0 = psm_ref[last * 3 + 2]
        dx = px - px0
        dy = py - py0
        dz = pz - pz0
        d = (dx * dx + dy * dy) + dz * dz
        md = jnp.minimum(mind_ref[...], d)
        mind_ref[...] = md
        rowmax = jnp.max(md, axis=1, keepdims=True)
        m = jnp.max(rowmax)
        rows_i = jax.lax.broadcasted_iota(jnp.int32, rowmax.shape, 0)
        r = jnp.min(jnp.where(rowmax == m, rows_i, jnp.int32(2147483647)))
        row = mind_ref[pl.ds(r, 1), :]
        cols_i = jax.lax.broadcasted_iota(jnp.int32, row.shape, 1)
        c = jnp.min(jnp.where(row == m, cols_i, jnp.int32(2147483647)))
        nxt = r * 128 + c
        out_ref[i + 1] = nxt
        return nxt

    jax.lax.fori_loop(0, ns - 1, body, jnp.int32(0))


def _fps(pos, n_samples):
    n = pos.shape[0]
    npad = -(-n // 128) * 128
    rows = npad // 128
    posp = jnp.pad(pos, ((0, npad - n), (0, 0)))
    pt = posp.T.reshape(3, rows, 128)
    return pl.pallas_call(
        functools.partial(_fps_body, n, n_samples),
        out_shape=jax.ShapeDtypeStruct((n_samples,), jnp.int32),
        in_specs=[pl.BlockSpec(memory_space=pltpu.VMEM)] * 3
        + [pl.BlockSpec(memory_space=pltpu.SMEM)],
        out_specs=pl.BlockSpec(memory_space=pltpu.SMEM),
        scratch_shapes=[pltpu.VMEM((rows, 128), jnp.float32)],
    )(pt[0], pt[1], pt[2], posp.reshape(-1))


def _radius(src_pos, q_pos, r, k):
    d2 = jnp.sum((q_pos[:, None, :] - src_pos[None, :, :]) ** 2, axis=-1)
    neg, nbr = jax.lax.top_k(-d2, k)
    valid = (-neg) <= r * r
    return nbr, valid


def _sc_topk_body(spad, qpad, nq, r2,
                  sxh, syh, szh, qxh, qyh, qzh, nbr_hbm,
                  sx, sy, sz, qx, qy, qz, d2s, cvals, cidx, outb):
    nch = spad // 16
    chsz = spad // 32
    inner = chsz // 16
    wid = jax.lax.axis_index("s") * 2 + jax.lax.axis_index("c")
    q0 = wid * nq
    pltpu.sync_copy(sxh, sx)
    pltpu.sync_copy(syh, sy)
    pltpu.sync_copy(szh, sz)
    pltpu.sync_copy(qxh.at[pl.ds(q0, nq)], qx.at[pl.ds(0, nq)])
    pltpu.sync_copy(qyh.at[pl.ds(q0, nq)], qy.at[pl.ds(0, nq)])
    pltpu.sync_copy(qzh.at[pl.ds(q0, nq)], qz.at[pl.ds(0, nq)])
    lane = jax.lax.iota(jnp.int32, 16)

    def vmin(v):
        for s in (8, 4, 2, 1):
            v = jnp.minimum(v, v.at[lane ^ s].get(mode="promise_in_bounds"))
        return v
    inf16 = jnp.full((16,), jnp.inf, dtype=jnp.float32)
    imax = jnp.int32(2147483647)
    imax16 = jnp.full((16,), imax, dtype=jnp.int32)

    def per_query(i, carry):
        qxs = qx[pl.ds(i, 16)][0]
        qys = qy[pl.ds(i, 16)][0]
        qzs = qz[pl.ds(i, 16)][0]

        def chunk_body(c, tau):
            def in_body(j, acc):
                b = c * chsz + j * 16
                dx = sx[pl.ds(b, 16)] - qxs
                dy = sy[pl.ds(b, 16)] - qys
                dz = sz[pl.ds(b, 16)] - qzs
                d2v = (dx * dx + dy * dy) + dz * dz
                d2s[pl.ds(b, 16)] = d2v
                return jnp.minimum(acc, d2v)

            acc = jax.lax.fori_loop(0, inner, in_body, inf16, unroll=inner)
            return jnp.maximum(tau, vmin(acc)[0])

        tau = jax.lax.fori_loop(0, 32, chunk_body, -jnp.inf)
        thr = jnp.minimum(tau, r2)

        def comp_body(j, off):
            v = d2s[pl.ds(j * 16, 16)]
            msk = v <= thr
            posv = off + plsc.cumsum(msk.astype(jnp.int32)) - 1
            plsc.store_scatter(cvals, [posv], v, mask=msk)
            plsc.store_scatter(cidx, [posv], j * 16 + lane, mask=msk)
            cnt = plsc.all_reduce_population_count(msk)[0]
            return off + cnt

        off = jax.lax.fori_loop(0, nch, comp_body, jnp.int32(0), unroll=8)
        cvals[pl.ds(off, 16)] = inf16
        cidx[pl.ds(off, 16)] = jnp.zeros((16,), jnp.int32)
        nsc = off // 16 + 1

        def sel_body(t, carry2):
            def scan_body(j, st):
                macc, iacc, pacc = st
                v = cvals[pl.ds(j * 16, 16)]
                ci = cidx[pl.ds(j * 16, 16)]
                pv = j * 16 + lane
                cond = (v < macc) | ((v == macc) & (ci < iacc))
                return (jnp.where(cond, v, macc),
                        jnp.where(cond, ci, iacc),
                        jnp.where(cond, pv, pacc))

            macc, iacc, pacc = jax.lax.fori_loop(
                0, nsc, scan_body, (inf16, imax16, imax16))
            m = vmin(macc)[0]
            bi = vmin(jnp.where(macc == m, iacc, imax))[0]
            p = vmin(jnp.where((macc == m) & (iacc == bi), pacc, imax))[0]
            lane0 = lane == 0
            plsc.store_scatter(outb, [jnp.full((16,), i * K + t, jnp.int32)],
                               jnp.full((16,), bi, jnp.int32), mask=lane0)
            plsc.store_scatter(cvals, [jnp.full((16,), p, jnp.int32)],
                               inf16, mask=lane0)
            return carry2

        jax.lax.fori_loop(0, K, sel_body, 0)
        return carry

    jax.lax.fori_loop(0, nq, per_query, 0)
    pltpu.sync_copy(outb, nbr_hbm.at[pl.ds(q0 * K, nq * K)])


def _sc_gather_body(rows_per_tec, table_hbm, nbr_hbm, out_hbm, idxv, rowsv, sem):
    wid = jax.lax.axis_index("s") * 2 + jax.lax.axis_index("c")
    base = wid * rows_per_tec

    def loop(j, c):
        b = base + j * 128
        pltpu.sync_copy(nbr_hbm.at[pl.ds(b, 128)], idxv)
        pltpu.async_copy(table_hbm.at[idxv], rowsv, sem).wait()
        pltpu.sync_copy(rowsv, out_hbm.at[pl.ds(b, 128)])
        return c

    jax.lax.fori_loop(0, rows_per_tec // 128, loop, 0)


def _sc_gather(table, nbr_flat):
    epad = nbr_flat.shape[0]
    d = table.shape[1]
    rows_per_tec = epad // 32
    mesh = plsc.VectorSubcoreMesh(core_axis_name="c", subcore_axis_name="s")
    f = functools.partial(
        pl.kernel,
        mesh=mesh,
        compiler_params=pltpu.CompilerParams(needs_layout_passes=False),
        out_type=jax.ShapeDtypeStruct((epad, d), jnp.float32),
        scratch_types=[
            pltpu.VMEM((128,), jnp.int32),
            pltpu.VMEM((128, d), jnp.float32),
            pltpu.SemaphoreType.DMA,
        ],
    )(functools.partial(_sc_gather_body, rows_per_tec))
    return f(table, nbr_flat)


def _table_body(x_ref, pos_ref, w_ref, b_ref, out_ref):
    y = jnp.dot(x_ref[...], w_ref[...], preferred_element_type=jnp.float32) + b_ref[...]
    rows = y.shape[0]
    pad = out_ref.shape[1] - y.shape[1] - 3
    out_ref[...] = jnp.concatenate(
        [y, pos_ref[...], jnp.zeros((rows, pad), jnp.float32)], axis=1)


def _table(x, posp, w, b, d):
    n = x.shape[0]
    blk = 512
    return pl.pallas_call(
        _table_body,
        grid=(n // blk,),
        in_specs=[
            pl.BlockSpec((blk, x.shape[1]), lambda i: (i, 0)),
            pl.BlockSpec((blk, 3), lambda i: (i, 0)),
            pl.BlockSpec(w.shape, lambda i: (0, 0)),
            pl.BlockSpec((1, w.shape[1]), lambda i: (0, 0)),
        ],
        out_specs=pl.BlockSpec((blk, d), lambda i: (i, 0)),
        out_shape=jax.ShapeDtypeStruct((n, d), jnp.float32),
    )(x, posp, w, b.reshape(1, -1))


def _conv_body(r2, qblk, ydim, edges_ref, qrep_ref, wp_ref, w2_ref, b2_ref, out_ref):
    blk = edges_ref[...]
    ye = blk[:, :ydim]
    posj = blk[:, ydim:ydim + 3]
    rel = posj - qrep_ref[...]
    d2 = jnp.sum(rel * rel, axis=1, keepdims=True)
    pterm = jnp.dot(rel, wp_ref[...], preferred_element_type=jnp.float32)
    h = jax.nn.sigmoid(ye + pterm)
    h = jnp.dot(h, w2_ref[...], preferred_element_type=jnp.float32) + b2_ref[...]
    od = h.shape[1]
    hm = jnp.where(d2 <= r2, h, -1e30)
    red = jnp.max(hm.reshape(qblk, K, od), axis=1)
    anyv = jnp.min(d2.reshape(qblk, K, 1), axis=1) <= r2
    out_ref[...] = jax.nn.relu(jnp.where(anyv, red, 0.0))


def _conv(edges, qrep, wp, w2, b2, r2, qpad, ydim):
    epad, d = edges.shape
    od = w2.shape[1]
    qblk = 128
    return pl.pallas_call(
        functools.partial(_conv_body, r2, qblk, ydim),
        grid=(qpad // qblk,),
        in_specs=[
            pl.BlockSpec((qblk * K, d), lambda i: (i, 0)),
            pl.BlockSpec((qblk * K, 3), lambda i: (i, 0)),
            pl.BlockSpec(wp.shape, lambda i: (0, 0)),
            pl.BlockSpec(w2.shape, lambda i: (0, 0)),
            pl.BlockSpec((1, od), lambda i: (0, 0)),
        ],
        out_specs=pl.BlockSpec((qblk, od), lambda i: (i, 0)),
        out_shape=jax.ShapeDtypeStruct((qpad, od), jnp.float32),
    )(edges, qrep, wp, w2, b2.reshape(1, -1))


def _radius_sc(src_pos, q_pos, r):
    ns = src_pos.shape[0]
    nq = q_pos.shape[0]
    spad = -(-ns // 512) * 512
    qpad = -(-nq // 256) * 256
    nq_tec = qpad // 32
    sposT = jnp.pad(src_pos, ((0, spad - ns), (0, 0)), constant_values=1e9).T
    qposT = jnp.pad(q_pos, ((0, qpad - nq), (0, 0)), constant_values=1e9).T
    mesh = plsc.VectorSubcoreMesh(core_axis_name="c", subcore_axis_name="s")
    f = functools.partial(
        pl.kernel,
        mesh=mesh,
        compiler_params=pltpu.CompilerParams(needs_layout_passes=False),
        out_type=jax.ShapeDtypeStruct((qpad * K,), jnp.int32),
        scratch_types=[
            pltpu.VMEM((spad,), jnp.float32),
            pltpu.VMEM((spad,), jnp.float32),
            pltpu.VMEM((spad,), jnp.float32),
            pltpu.VMEM((nq_tec + 16,), jnp.float32),
            pltpu.VMEM((nq_tec + 16,), jnp.float32),
            pltpu.VMEM((nq_tec + 16,), jnp.float32),
            pltpu.VMEM((spad,), jnp.float32),
            pltpu.VMEM((spad + 16,), jnp.float32),
            pltpu.VMEM((spad + 16,), jnp.int32),
            pltpu.VMEM((nq_tec * K,), jnp.int32),
        ],
    )(functools.partial(_sc_topk_body, spad, qpad, nq_tec, jnp.float32(r * r)))
    return f(sposT[0], sposT[1], sposT[2],
             qposT[0], qposT[1], qposT[2]), qpad


def _pn_conv(x_src, src_pos, q_pos, nbr, valid, w1, b1, w2, b2):
    x_j = x_src[nbr]
    rel = src_pos[nbr] - q_pos[:, None, :]
    h = jnp.concatenate([x_j, rel], axis=-1)
    h = jax.nn.sigmoid(h @ w1 + b1) @ w2 + b2
    h = jnp.where(valid[:, :, None], h, -1e30)
    out = jnp.max(h, axis=1)
    out = jnp.where(jnp.any(valid, axis=1, keepdims=True), out, 0.0)
    return jax.nn.relu(out)


def _tail_body(nrows, x2p_ref, pos2p_ref, wga_ref, bga_ref, wgb_ref, bgb_ref,
               wfa_ref, bfa_ref, wfb_ref, bfb_ref, out_ref):
    g = jnp.concatenate([x2p_ref[...], pos2p_ref[...]], axis=-1)
    h = jax.nn.sigmoid(
        jnp.dot(g, wga_ref[...], preferred_element_type=jnp.float32) + bga_ref[...])
    h = jnp.dot(h, wgb_ref[...], preferred_element_type=jnp.float32) + bgb_ref[...]
    h = jax.nn.relu(h)
    rows = jax.lax.broadcasted_iota(jnp.int32, h.shape, 0)
    h = jnp.where(rows < nrows, h, -jnp.inf)
    pooled = jnp.max(h, axis=0, keepdims=True)
    o = jax.nn.sigmoid(
        jnp.dot(pooled, wfa_ref[...], preferred_element_type=jnp.float32) + bfa_ref[...])
    o = jnp.dot(o, wfb_ref[...], preferred_element_type=jnp.float32) + bfb_ref[...]
    out_ref[...] = jax.nn.relu(o)


def _tail(x2, pos2, wga, bga, wgb, bgb, wfa, bfa, wfb, bfb):
    n = x2.shape[0]
    npad = (n + 7) // 8 * 8
    x2p = jnp.pad(x2, ((0, npad - n), (0, 0)))
    pos2p = jnp.pad(pos2, ((0, npad - n), (0, 0)))
    return pl.pallas_call(
        functools.partial(_tail_body, n),
        out_shape=jax.ShapeDtypeStruct((1, 128), jnp.float32),
    )(x2p, pos2p, wga, bga.reshape(1, -1), wgb, bgb.reshape(1, -1),
      wfa, bfa.reshape(1, -1), wfb, bfb.reshape(1, -1))


def kernel(x, pos, batch, w1a, b1a, w1b, b1b, w2a, b2a, w2b, b2b,
           wga, bga, wgb, bgb, wfa, bfa, wfb, bfb):
    idx1 = _fps(pos, NS1)
    pos1 = pos[idx1]
    nbrf1, qpad1 = _radius_sc(pos, pos1, 2.0)

    posp = jnp.pad(pos, ((0, 10240 - N), (0, 0)))
    xp = jnp.pad(x, ((0, 10240 - N), (0, 0)))
    table1 = _table(xp, posp, w1a[:C], b1a, 128)
    edges1 = _sc_gather(table1, nbrf1)
    pos1p = jnp.pad(pos1, ((0, qpad1 - NS1), (0, 0)))
    qrep1 = jnp.broadcast_to(pos1p[:, None, :], (qpad1, K, 3)).reshape(qpad1 * K, 3)
    x1 = _conv(edges1, qrep1, w1a[C:], w1b, b1b, 4.0, qpad1, 64)

    idx2 = _fps(pos1, NS2)
    pos2 = pos1[idx2]
    nbrf2, qpad2 = _radius_sc(pos1, pos2, 4.0)

    table2 = _table(x1, pos1p, w2a[:128], b2a, 256)
    edges2 = _sc_gather(table2, nbrf2)
    pos2p = jnp.pad(pos2, ((0, qpad2 - NS2), (0, 0)))
    qrep2 = jnp.broadcast_to(pos2p[:, None, :], (qpad2, K, 3)).reshape(qpad2 * K, 3)
    x2 = _conv(edges2, qrep2, w2a[128:], w2b, b2b, 16.0, qpad2, 128)

    return _tail(x2[:NS2], pos2, wga, bga, wgb, bgb, wfa, bfa, wfb, bfb)


# speedup vs baseline: 1.0485x; 1.0485x over previous
import functools

import jax
import jax.numpy as jnp
from jax.experimental import pallas as pl
from jax.experimental.pallas import tpu as pltpu
from jax.experimental.pallas import tpu_sc as plsc

N = 10000
C = 256
K = 32
NS1 = 2500
NS2 = 625


def _fps_body(nreal, ns, px_ref, py_ref, pz_ref, psm_ref, out_ref, mind_ref):
    rows = px_ref.shape[0]
    flat = (jax.lax.broadcasted_iota(jnp.int32, (rows, 128), 0) * 128
            + jax.lax.broadcasted_iota(jnp.int32, (rows, 128), 1))
    mind_ref[...] = jnp.where(flat < nreal, jnp.inf, -jnp.inf)
    out_ref[0] = 0
    px = px_ref[...]
    py = py_ref[...]
    pz = pz_ref[...]

    def body(i, last):
        px0 = psm_ref[last * 3]
        py0 = psm_ref[last * 3 + 1]
        pz0 = psm_ref[last * 3 + 2]
        dx = px - px0
        dy = py - py0
        dz = pz - pz0
        d = (dx * dx + dy * dy) + dz * dz
        md = jnp.minimum(mind_ref[...], d)
        mind_ref[...] = md
        m = jnp.max(md)
        cand = jnp.where(md == m, flat, jnp.int32(2147483647))
        nxt = jnp.min(cand)
        out_ref[i + 1] = nxt
        return nxt

    jax.lax.fori_loop(0, ns - 1, body, jnp.int32(0))


def _fps(pos, n_samples):
    n = pos.shape[0]
    npad = -(-n // 128) * 128
    rows = npad // 128
    posp = jnp.pad(pos, ((0, npad - n), (0, 0)))
    pt = posp.T.reshape(3, rows, 128)
    return pl.pallas_call(
        functools.partial(_fps_body, n, n_samples),
        out_shape=jax.ShapeDtypeStruct((n_samples,), jnp.int32),
        in_specs=[pl.BlockSpec(memory_space=pltpu.VMEM)] * 3
        + [pl.BlockSpec(memory_space=pltpu.SMEM)],
        out_specs=pl.BlockSpec(memory_space=pltpu.SMEM),
        scratch_shapes=[pltpu.VMEM((rows, 128), jnp.float32)],
    )(pt[0], pt[1], pt[2], posp.reshape(-1))


def _sc_topk_body(spad, qpad, nq, r2,
                  sxh, syh, szh, qxh, qyh, qzh, nbr_hbm,
                  sx, sy, sz, qx, qy, qz, d2s, cvals, cidx, outb):
    nch = spad // 16
    chsz = spad // 32
    inner = chsz // 16
    wid = jax.lax.axis_index("s") * 2 + jax.lax.axis_index("c")
    q0 = wid * nq
    pltpu.sync_copy(sxh, sx)
    pltpu.sync_copy(syh, sy)
    pltpu.sync_copy(szh, sz)
    pltpu.sync_copy(qxh.at[pl.ds(q0, nq)], qx.at[pl.ds(0, nq)])
    pltpu.sync_copy(qyh.at[pl.ds(q0, nq)], qy.at[pl.ds(0, nq)])
    pltpu.sync_copy(qzh.at[pl.ds(q0, nq)], qz.at[pl.ds(0, nq)])
    lane = jax.lax.iota(jnp.int32, 16)

    def vmin(v):
        for s in (8, 4, 2, 1):
            v = jnp.minimum(v, v.at[lane ^ s].get(mode="promise_in_bounds"))
        return v
    inf16 = jnp.full((16,), jnp.inf, dtype=jnp.float32)
    imax = jnp.int32(2147483647)
    imax16 = jnp.full((16,), imax, dtype=jnp.int32)

    def per_query(i, carry):
        qxs = qx[pl.ds(i, 16)][0]
        qys = qy[pl.ds(i, 16)][0]
        qzs = qz[pl.ds(i, 16)][0]

        def chunk_body(c, tau):
            def in_body(j, acc):
                b = c * chsz + j * 16
                dx = sx[pl.ds(b, 16)] - qxs
                dy = sy[pl.ds(b, 16)] - qys
                dz = sz[pl.ds(b, 16)] - qzs
                d2v = (dx * dx + dy * dy) + dz * dz
                d2s[pl.ds(b, 16)] = d2v
                return jnp.minimum(acc, d2v)

            acc = jax.lax.fori_loop(0, inner, in_body, inf16, unroll=inner)
            return jnp.maximum(tau, vmin(acc)[0])

        tau = jax.lax.fori_loop(0, 32, chunk_body, -jnp.inf)
        thr = jnp.minimum(tau, r2)

        def comp_body(j, off):
            v = d2s[pl.ds(j * 16, 16)]
            msk = v <= thr
            posv = off + plsc.cumsum(msk.astype(jnp.int32)) - 1
            plsc.store_scatter(cvals, [posv], v, mask=msk)
            plsc.store_scatter(cidx, [posv], j * 16 + lane, mask=msk)
            cnt = plsc.all_reduce_population_count(msk)[0]
            return off + cnt

        off = jax.lax.fori_loop(0, nch, comp_body, jnp.int32(0), unroll=4)
        cvals[pl.ds(off, 16)] = inf16
        cidx[pl.ds(off, 16)] = jnp.zeros((16,), jnp.int32)
        nsc = off // 16 + 1

        def sel_body(t, carry2):
            def scan_body(j, st):
                macc, iacc, pacc = st
                v = cvals[pl.ds(j * 16, 16)]
                ci = cidx[pl.ds(j * 16, 16)]
                pv = j * 16 + lane
                cond = (v < macc) | ((v == macc) & (ci < iacc))
                return (jnp.where(cond, v, macc),
                        jnp.where(cond, ci, iacc),
                        jnp.where(cond, pv, pacc))

            macc, iacc, pacc = jax.lax.fori_loop(
                0, nsc, scan_body, (inf16, imax16, imax16))
            m = vmin(macc)[0]
            bi = vmin(jnp.where(macc == m, iacc, imax))[0]
            p = vmin(jnp.where((macc == m) & (iacc == bi), pacc, imax))[0]
            lane0 = lane == 0
            plsc.store_scatter(outb, [jnp.full((16,), i * K + t, jnp.int32)],
                               jnp.full((16,), bi, jnp.int32), mask=lane0)
            plsc.store_scatter(cvals, [jnp.full((16,), p, jnp.int32)],
                               inf16, mask=lane0)
            return carry2

        jax.lax.fori_loop(0, K, sel_body, 0)
        return carry

    jax.lax.fori_loop(0, nq, per_query, 0)
    pltpu.sync_copy(outb, nbr_hbm.at[pl.ds(q0 * K, nq * K)])


def _sc_gather_body(rows_per_tec, table_hbm, nbr_hbm, out_hbm, idxv, rowsv, sem):
    wid = jax.lax.axis_index("s") * 2 + jax.lax.axis_index("c")
    base = wid * rows_per_tec

    def loop(j, c):
        b = base + j * 128
        pltpu.sync_copy(nbr_hbm.at[pl.ds(b, 128)], idxv)
        pltpu.async_copy(table_hbm.at[idxv], rowsv, sem).wait()
        pltpu.sync_copy(rowsv, out_hbm.at[pl.ds(b, 128)])
        return c

    jax.lax.fori_loop(0, rows_per_tec // 128, loop, 0)


def _sc_gather(table, nbr_flat):
    epad = nbr_flat.shape[0]
    d = table.shape[1]
    rows_per_tec = epad // 32
    mesh = plsc.VectorSubcoreMesh(core_axis_name="c", subcore_axis_name="s")
    f = functools.partial(
        pl.kernel,
        mesh=mesh,
        compiler_params=pltpu.CompilerParams(needs_layout_passes=False),
        out_type=jax.ShapeDtypeStruct((epad, d), jnp.float32),
        scratch_types=[
            pltpu.VMEM((128,), jnp.int32),
            pltpu.VMEM((128, d), jnp.float32),
            pltpu.SemaphoreType.DMA,
        ],
    )(functools.partial(_sc_gather_body, rows_per_tec))
    return f(table, nbr_flat)


def _table_body(x_ref, pos_ref, w_ref, b_ref, out_ref):
    y = jnp.dot(x_ref[...], w_ref[...], preferred_element_type=jnp.float32) + b_ref[...]
    rows = y.shape[0]
    pad = out_ref.shape[1] - y.shape[1] - 3
    out_ref[...] = jnp.concatenate(
        [y, pos_ref[...], jnp.zeros((rows, pad), jnp.float32)], axis=1)


def _table(x, posp, w, b, d):
    n = x.shape[0]
    blk = 512
    return pl.pallas_call(
        _table_body,
        grid=(n // blk,),
        in_specs=[
            pl.BlockSpec((blk, x.shape[1]), lambda i: (i, 0)),
            pl.BlockSpec((blk, 3), lambda i: (i, 0)),
            pl.BlockSpec(w.shape, lambda i: (0, 0)),
            pl.BlockSpec((1, w.shape[1]), lambda i: (0, 0)),
        ],
        out_specs=pl.BlockSpec((blk, d), lambda i: (i, 0)),
        out_shape=jax.ShapeDtypeStruct((n, d), jnp.float32),
    )(x, posp, w, b.reshape(1, -1))


def _conv_body(r2, qblk, ydim, edges_ref, qrep_ref, wp_ref, w2_ref, b2_ref, out_ref):
    blk = edges_ref[...]
    ye = blk[:, :ydim]
    posj = blk[:, ydim:ydim + 3]
    rel = posj - qrep_ref[...]
    d2 = jnp.sum(rel * rel, axis=1, keepdims=True)
    pterm = jnp.dot(rel, wp_ref[...], preferred_element_type=jnp.float32)
    h = jax.nn.sigmoid(ye + pterm)
    h = jnp.dot(h, w2_ref[...], preferred_element_type=jnp.float32) + b2_ref[...]
    od = h.shape[1]
    hm = jnp.where(d2 <= r2, h, -1e30)
    red = jnp.max(hm.reshape(qblk, K, od), axis=1)
    anyv = jnp.min(d2.reshape(qblk, K, 1), axis=1) <= r2
    out_ref[...] = jax.nn.relu(jnp.where(anyv, red, 0.0))


def _conv(edges, qrep, wp, w2, b2, r2, qpad, ydim):
    epad, d = edges.shape
    od = w2.shape[1]
    qblk = 128
    return pl.pallas_call(
        functools.partial(_conv_body, r2, qblk, ydim),
        grid=(qpad // qblk,),
        in_specs=[
            pl.BlockSpec((qblk * K, d), lambda i: (i, 0)),
            pl.BlockSpec((qblk * K, 3), lambda i: (i, 0)),
            pl.BlockSpec(wp.shape, lambda i: (0, 0)),
            pl.BlockSpec(w2.shape, lambda i: (0, 0)),
            pl.BlockSpec((1, od), lambda i: (0, 0)),
        ],
        out_specs=pl.BlockSpec((qblk, od), lambda i: (i, 0)),
        out_shape=jax.ShapeDtypeStruct((qpad, od), jnp.float32),
    )(edges, qrep, wp, w2, b2.reshape(1, -1))


def _radius_sc(src_pos, q_pos, r):
    ns = src_pos.shape[0]
    nq = q_pos.shape[0]
    spad = -(-ns // 512) * 512
    qpad = -(-nq // 256) * 256
    nq_tec = qpad // 32
    sposT = jnp.pad(src_pos, ((0, spad - ns), (0, 0)), constant_values=1e9).T
    qposT = jnp.pad(q_pos, ((0, qpad - nq), (0, 0)), constant_values=1e9).T
    mesh = plsc.VectorSubcoreMesh(core_axis_name="c", subcore_axis_name="s")
    f = functools.partial(
        pl.kernel,
        mesh=mesh,
        compiler_params=pltpu.CompilerParams(needs_layout_passes=False),
        out_type=jax.ShapeDtypeStruct((qpad * K,), jnp.int32),
        scratch_types=[
            pltpu.VMEM((spad,), jnp.float32),
            pltpu.VMEM((spad,), jnp.float32),
            pltpu.VMEM((spad,), jnp.float32),
            pltpu.VMEM((nq_tec + 16,), jnp.float32),
            pltpu.VMEM((nq_tec + 16,), jnp.float32),
            pltpu.VMEM((nq_tec + 16,), jnp.float32),
            pltpu.VMEM((spad,), jnp.float32),
            pltpu.VMEM((spad + 16,), jnp.float32),
            pltpu.VMEM((spad + 16,), jnp.int32),
            pltpu.VMEM((nq_tec * K,), jnp.int32),
        ],
    )(functools.partial(_sc_topk_body, spad, qpad, nq_tec, jnp.float32(r * r)))
    return f(sposT[0], sposT[1], sposT[2],
             qposT[0], qposT[1], qposT[2]), qpad


def _tail_body(nrows, x2p_ref, pos2p_ref, wga_ref, bga_ref, wgb_ref, bgb_ref,
               wfa_ref, bfa_ref, wfb_ref, bfb_ref, out_ref):
    g = jnp.concatenate([x2p_ref[...], pos2p_ref[...]], axis=-1)
    h = jax.nn.sigmoid(
        jnp.dot(g, wga_ref[...], preferred_element_type=jnp.float32) + bga_ref[...])
    h = jnp.dot(h, wgb_ref[...], preferred_element_type=jnp.float32) + bgb_ref[...]
    h = jax.nn.relu(h)
    rows = jax.lax.broadcasted_iota(jnp.int32, h.shape, 0)
    h = jnp.where(rows < nrows, h, -jnp.inf)
    pooled = jnp.max(h, axis=0, keepdims=True)
    o = jax.nn.sigmoid(
        jnp.dot(pooled, wfa_ref[...], preferred_element_type=jnp.float32) + bfa_ref[...])
    o = jnp.dot(o, wfb_ref[...], preferred_element_type=jnp.float32) + bfb_ref[...]
    out_ref[...] = jax.nn.relu(o)


def _tail(x2, pos2, wga, bga, wgb, bgb, wfa, bfa, wfb, bfb):
    n = x2.shape[0]
    npad = (n + 7) // 8 * 8
    x2p = jnp.pad(x2, ((0, npad - n), (0, 0)))
    pos2p = jnp.pad(pos2, ((0, npad - n), (0, 0)))
    return pl.pallas_call(
        functools.partial(_tail_body, n),
        out_shape=jax.ShapeDtypeStruct((1, 128), jnp.float32),
    )(x2p, pos2p, wga, bga.reshape(1, -1), wgb, bgb.reshape(1, -1),
      wfa, bfa.reshape(1, -1), wfb, bfb.reshape(1, -1))


def kernel(x, pos, batch, w1a, b1a, w1b, b1b, w2a, b2a, w2b, b2b,
           wga, bga, wgb, bgb, wfa, bfa, wfb, bfb):
    idx1 = _fps(pos, NS1)
    pos1 = pos[idx1]
    nbrf1, qpad1 = _radius_sc(pos, pos1, 2.0)

    posp = jnp.pad(pos, ((0, 10240 - N), (0, 0)))
    xp = jnp.pad(x, ((0, 10240 - N), (0, 0)))
    table1 = _table(xp, posp, w1a[:C], b1a, 128)
    edges1 = _sc_gather(table1, nbrf1)
    pos1p = jnp.pad(pos1, ((0, qpad1 - NS1), (0, 0)))
    qrep1 = jnp.broadcast_to(pos1p[:, None, :], (qpad1, K, 3)).reshape(qpad1 * K, 3)
    x1 = _conv(edges1, qrep1, w1a[C:], w1b, b1b, 4.0, qpad1, 64)

    idx2 = _fps(pos1, NS2)
    pos2 = pos1[idx2]
    nbrf2, qpad2 = _radius_sc(pos1, pos2, 4.0)

    table2 = _table(x1, pos1p, w2a[:128], b2a, 256)
    edges2 = _sc_gather(table2, nbrf2)
    pos2p = jnp.pad(pos2, ((0, qpad2 - NS2), (0, 0)))
    qrep2 = jnp.broadcast_to(pos2p[:, None, :], (qpad2, K, 3)).reshape(qpad2 * K, 3)
    x2 = _conv(edges2, qrep2, w2a[128:], w2b, b2b, 16.0, qpad2, 128)

    return _tail(x2[:NS2], pos2, wga, bga, wgb, bgb, wfa, bfa, wfb, bfb)


# speedup vs baseline: 1.0494x; 1.0009x over previous
import functools

import jax
import jax.numpy as jnp
from jax.experimental import pallas as pl
from jax.experimental.pallas import tpu as pltpu
from jax.experimental.pallas import tpu_sc as plsc

N = 10000
C = 256
K = 32
NS1 = 2500
NS2 = 625


def _fps_body(nreal, ns, px_ref, py_ref, pz_ref, psm_ref, out_ref, mind_ref):
    rows = px_ref.shape[0]
    flat = (jax.lax.broadcasted_iota(jnp.int32, (rows, 128), 0) * 128
            + jax.lax.broadcasted_iota(jnp.int32, (rows, 128), 1))
    mind_ref[...] = jnp.where(flat < nreal, jnp.inf, -jnp.inf)
    out_ref[0] = 0
    px = px_ref[...]
    py = py_ref[...]
    pz = pz_ref[...]

    def body(i, last):
        px0 = psm_ref[last * 3]
        py0 = psm_ref[last * 3 + 1]
        pz0 = psm_ref[last * 3 + 2]
        dx = px - px0
        dy = py - py0
        dz = pz - pz0
        d = (dx * dx + dy * dy) + dz * dz
        md = jnp.minimum(mind_ref[...], d)
        mind_ref[...] = md
        m = jnp.max(md)
        cand = jnp.where(md == m, flat, jnp.int32(2147483647))
        nxt = jnp.min(cand)
        out_ref[i + 1] = nxt
        return nxt

    jax.lax.fori_loop(0, ns - 1, body, jnp.int32(0))


def _fps(pos, n_samples):
    n = pos.shape[0]
    npad = -(-n // 128) * 128
    rows = npad // 128
    posp = jnp.pad(pos, ((0, npad - n), (0, 0)))
    pt = posp.T.reshape(3, rows, 128)
    return pl.pallas_call(
        functools.partial(_fps_body, n, n_samples),
        out_shape=jax.ShapeDtypeStruct((n_samples,), jnp.int32),
        in_specs=[pl.BlockSpec(memory_space=pltpu.VMEM)] * 3
        + [pl.BlockSpec(memory_space=pltpu.SMEM)],
        out_specs=pl.BlockSpec(memory_space=pltpu.SMEM),
        scratch_shapes=[pltpu.VMEM((rows, 128), jnp.float32)],
    )(pt[0], pt[1], pt[2], posp.reshape(-1))


def _sc_topk_body(spad, qpad, nq, r2,
                  sxh, syh, szh, qxh, qyh, qzh, nbr_hbm,
                  sx, sy, sz, qx, qy, qz, d2s, cvals, cidx, outb):
    nch = spad // 16
    chsz = spad // 32
    inner = chsz // 16
    wid = jax.lax.axis_index("s") * 2 + jax.lax.axis_index("c")
    q0 = wid * nq
    pltpu.sync_copy(sxh, sx)
    pltpu.sync_copy(syh, sy)
    pltpu.sync_copy(szh, sz)
    pltpu.sync_copy(qxh.at[pl.ds(q0, nq)], qx.at[pl.ds(0, nq)])
    pltpu.sync_copy(qyh.at[pl.ds(q0, nq)], qy.at[pl.ds(0, nq)])
    pltpu.sync_copy(qzh.at[pl.ds(q0, nq)], qz.at[pl.ds(0, nq)])
    lane = jax.lax.iota(jnp.int32, 16)

    def vmin(v):
        for s in (8, 4, 2, 1):
            v = jnp.minimum(v, v.at[lane ^ s].get(mode="promise_in_bounds"))
        return v
    inf16 = jnp.full((16,), jnp.inf, dtype=jnp.float32)
    imax = jnp.int32(2147483647)
    imax16 = jnp.full((16,), imax, dtype=jnp.int32)

    def per_query(i, carry):
        qxs = qx[pl.ds(i, 16)][0]
        qys = qy[pl.ds(i, 16)][0]
        qzs = qz[pl.ds(i, 16)][0]

        def chunk_body(c, tau):
            def in_body(j, acc):
                b = c * chsz + j * 16
                dx = sx[pl.ds(b, 16)] - qxs
                dy = sy[pl.ds(b, 16)] - qys
                dz = sz[pl.ds(b, 16)] - qzs
                d2v = (dx * dx + dy * dy) + dz * dz
                d2s[pl.ds(b, 16)] = d2v
                return jnp.minimum(acc, d2v)

            acc = jax.lax.fori_loop(0, inner, in_body, inf16, unroll=inner)
            return jnp.maximum(tau, vmin(acc)[0])

        tau = jax.lax.fori_loop(0, 32, chunk_body, -jnp.inf)
        thr = jnp.minimum(tau, r2)

        def comp_body(j, off):
            v = d2s[pl.ds(j * 16, 16)]
            msk = v <= thr
            posv = off + plsc.cumsum(msk.astype(jnp.int32)) - 1
            plsc.store_scatter(cvals, [posv], v, mask=msk)
            plsc.store_scatter(cidx, [posv], j * 16 + lane, mask=msk)
            cnt = plsc.all_reduce_population_count(msk)[0]
            return off + cnt

        off = jax.lax.fori_loop(0, nch, comp_body, jnp.int32(0), unroll=4)
        cvals[pl.ds(off, 16)] = inf16
        cidx[pl.ds(off, 16)] = jnp.zeros((16,), jnp.int32)
        nsc = off // 16 + 1

        def sel_body(t, carry2):
            def scan_body(j, st):
                macc, iacc, pacc = st
                v = cvals[pl.ds(j * 16, 16)]
                ci = cidx[pl.ds(j * 16, 16)]
                pv = j * 16 + lane
                cond = (v < macc) | ((v == macc) & (ci < iacc))
                return (jnp.where(cond, v, macc),
                        jnp.where(cond, ci, iacc),
                        jnp.where(cond, pv, pacc))

            macc, iacc, pacc = jax.lax.fori_loop(
                0, nsc, scan_body, (inf16, imax16, imax16))
            m = vmin(macc)[0]
            bi = vmin(jnp.where(macc == m, iacc, imax))[0]
            p = vmin(jnp.where((macc == m) & (iacc == bi), pacc, imax))[0]
            lane0 = lane == 0
            plsc.store_scatter(outb, [jnp.full((16,), i * K + t, jnp.int32)],
                               jnp.full((16,), bi, jnp.int32), mask=lane0)
            plsc.store_scatter(cvals, [jnp.full((16,), p, jnp.int32)],
                               inf16, mask=lane0)
            return carry2

        jax.lax.fori_loop(0, K, sel_body, 0)
        return carry

    jax.lax.fori_loop(0, nq, per_query, 0)
    pltpu.sync_copy(outb, nbr_hbm.at[pl.ds(q0 * K, nq * K)])


def _sc_gather_body(rows_per_tec, table_hbm, nbr_hbm, out_hbm, idxv, rowsv, sem):
    wid = jax.lax.axis_index("s") * 2 + jax.lax.axis_index("c")
    base = wid * rows_per_tec

    def loop(j, c):
        b = base + j * 128
        pltpu.sync_copy(nbr_hbm.at[pl.ds(b, 128)], idxv)
        pltpu.async_copy(table_hbm.at[idxv], rowsv, sem).wait()
        pltpu.sync_copy(rowsv, out_hbm.at[pl.ds(b, 128)])
        return c

    jax.lax.fori_loop(0, rows_per_tec // 128, loop, 0)


def _sc_gather(table, nbr_flat):
    epad = nbr_flat.shape[0]
    d = table.shape[1]
    rows_per_tec = epad // 32
    mesh = plsc.VectorSubcoreMesh(core_axis_name="c", subcore_axis_name="s")
    f = functools.partial(
        pl.kernel,
        mesh=mesh,
        compiler_params=pltpu.CompilerParams(needs_layout_passes=False),
        out_type=jax.ShapeDtypeStruct((epad, d), jnp.float32),
        scratch_types=[
            pltpu.VMEM((128,), jnp.int32),
            pltpu.VMEM((128, d), jnp.float32),
            pltpu.SemaphoreType.DMA,
        ],
    )(functools.partial(_sc_gather_body, rows_per_tec))
    return f(table, nbr_flat)


def _table_body(x_ref, pos_ref, w_ref, b_ref, out_ref):
    y = jnp.dot(x_ref[...], w_ref[...], preferred_element_type=jnp.float32) + b_ref[...]
    rows = y.shape[0]
    pad = out_ref.shape[1] - y.shape[1] - 3
    out_ref[...] = jnp.concatenate(
        [y, pos_ref[...], jnp.zeros((rows, pad), jnp.float32)], axis=1)


def _table(x, posp, w, b, d):
    n = x.shape[0]
    blk = 512
    return pl.pallas_call(
        _table_body,
        grid=(n // blk,),
        in_specs=[
            pl.BlockSpec((blk, x.shape[1]), lambda i: (i, 0)),
            pl.BlockSpec((blk, 3), lambda i: (i, 0)),
            pl.BlockSpec(w.shape, lambda i: (0, 0)),
            pl.BlockSpec((1, w.shape[1]), lambda i: (0, 0)),
        ],
        out_specs=pl.BlockSpec((blk, d), lambda i: (i, 0)),
        out_shape=jax.ShapeDtypeStruct((n, d), jnp.float32),
    )(x, posp, w, b.reshape(1, -1))


def _conv_body(r2, qblk, ydim, edges_ref, qrep_ref, wp_ref, w2_ref, b2_ref, out_ref):
    blk = edges_ref[...]
    ye = blk[:, :ydim]
    posj = blk[:, ydim:ydim + 3]
    rel = posj - qrep_ref[...]
    d2 = jnp.sum(rel * rel, axis=1, keepdims=True)
    pterm = jnp.dot(rel, wp_ref[...], preferred_element_type=jnp.float32)
    h = jax.nn.sigmoid(ye + pterm)
    h = jnp.dot(h, w2_ref[...], preferred_element_type=jnp.float32) + b2_ref[...]
    od = h.shape[1]
    hm = jnp.where(d2 <= r2, h, -1e30)
    red = jnp.max(hm.reshape(qblk, K, od), axis=1)
    anyv = jnp.min(d2.reshape(qblk, K, 1), axis=1) <= r2
    out_ref[...] = jax.nn.relu(jnp.where(anyv, red, 0.0))


def _conv(edges, qrep, wp, w2, b2, r2, qpad, ydim):
    epad, d = edges.shape
    od = w2.shape[1]
    qblk = 128
    return pl.pallas_call(
        functools.partial(_conv_body, r2, qblk, ydim),
        grid=(qpad // qblk,),
        in_specs=[
            pl.BlockSpec((qblk * K, d), lambda i: (i, 0)),
            pl.BlockSpec((qblk * K, 3), lambda i: (i, 0)),
            pl.BlockSpec(wp.shape, lambda i: (0, 0)),
            pl.BlockSpec(w2.shape, lambda i: (0, 0)),
            pl.BlockSpec((1, od), lambda i: (0, 0)),
        ],
        out_specs=pl.BlockSpec((qblk, od), lambda i: (i, 0)),
        out_shape=jax.ShapeDtypeStruct((qpad, od), jnp.float32),
    )(edges, qrep, wp, w2, b2.reshape(1, -1))


def _radius_sc(src_pos, q_pos, r):
    ns = src_pos.shape[0]
    nq = q_pos.shape[0]
    spad = -(-ns // 512) * 512
    qpad = -(-nq // 256) * 256
    nq_tec = qpad // 32
    sposT = jnp.pad(src_pos, ((0, spad - ns), (0, 0)), constant_values=1e9).T
    qposT = jnp.pad(q_pos, ((0, qpad - nq), (0, 0)), constant_values=1e9).T
    mesh = plsc.VectorSubcoreMesh(core_axis_name="c", subcore_axis_name="s")
    f = functools.partial(
        pl.kernel,
        mesh=mesh,
        compiler_params=pltpu.CompilerParams(needs_layout_passes=False),
        out_type=jax.ShapeDtypeStruct((qpad * K,), jnp.int32),
        scratch_types=[
            pltpu.VMEM((spad,), jnp.float32),
            pltpu.VMEM((spad,), jnp.float32),
            pltpu.VMEM((spad,), jnp.float32),
            pltpu.VMEM((nq_tec + 16,), jnp.float32),
            pltpu.VMEM((nq_tec + 16,), jnp.float32),
            pltpu.VMEM((nq_tec + 16,), jnp.float32),
            pltpu.VMEM((spad,), jnp.float32),
            pltpu.VMEM((spad + 16,), jnp.float32),
            pltpu.VMEM((spad + 16,), jnp.int32),
            pltpu.VMEM((nq_tec * K,), jnp.int32),
        ],
    )(functools.partial(_sc_topk_body, spad, qpad, nq_tec, jnp.float32(r * r)))
    return f(sposT[0], sposT[1], sposT[2],
             qposT[0], qposT[1], qposT[2]), qpad


def _tail_body(nrows, x2p_ref, pos2p_ref, wga_ref, bga_ref, wgb_ref, bgb_ref,
               wfa_ref, bfa_ref, wfb_ref, bfb_ref, out_ref):
    g = jnp.concatenate([x2p_ref[...], pos2p_ref[...]], axis=-1)
    h = jax.nn.sigmoid(
        jnp.dot(g, wga_ref[...], preferred_element_type=jnp.float32) + bga_ref[...])
    h = jnp.dot(h, wgb_ref[...], preferred_element_type=jnp.float32) + bgb_ref[...]
    h = jax.nn.relu(h)
    rows = jax.lax.broadcasted_iota(jnp.int32, h.shape, 0)
    h = jnp.where(rows < nrows, h, -jnp.inf)
    pooled = jnp.max(h, axis=0, keepdims=True)
    o = jax.nn.sigmoid(
        jnp.dot(pooled, wfa_ref[...], preferred_element_type=jnp.float32) + bfa_ref[...])
    o = jnp.dot(o, wfb_ref[...], preferred_element_type=jnp.float32) + bfb_ref[...]
    out_ref[...] = jax.nn.relu(o)


def _tail(x2, pos2, wga, bga, wgb, bgb, wfa, bfa, wfb, bfb):
    n = x2.shape[0]
    npad = (n + 7) // 8 * 8
    x2p = jnp.pad(x2, ((0, npad - n), (0, 0)))
    pos2p = jnp.pad(pos2, ((0, npad - n), (0, 0)))
    return pl.pallas_call(
        functools.partial(_tail_body, n),
        out_shape=jax.ShapeDtypeStruct((1, 128), jnp.float32),
    )(x2p, pos2p, wga, bga.reshape(1, -1), wgb, bgb.reshape(1, -1),
      wfa, bfa.reshape(1, -1), wfb, bfb.reshape(1, -1))


def kernel(x, pos, batch, w1a, b1a, w1b, b1b, w2a, b2a, w2b, b2b,
           wga, bga, wgb, bgb, wfa, bfa, wfb, bfb):
    idx1 = _fps(pos, NS1)
    pos1 = pos[idx1]
    idx2 = _fps(pos1, NS2)
    pos2 = pos1[idx2]
    nbrf1, qpad1 = _radius_sc(pos, pos1, 2.0)
    nbrf2, qpad2 = _radius_sc(pos1, pos2, 4.0)

    posp = jnp.pad(pos, ((0, 10240 - N), (0, 0)))
    xp = jnp.pad(x, ((0, 10240 - N), (0, 0)))
    table1 = _table(xp, posp, w1a[:C], b1a, 128)
    edges1 = _sc_gather(table1, nbrf1)
    pos1p = jnp.pad(pos1, ((0, qpad1 - NS1), (0, 0)))
    qrep1 = jnp.broadcast_to(pos1p[:, None, :], (qpad1, K, 3)).reshape(qpad1 * K, 3)
    x1 = _conv(edges1, qrep1, w1a[C:], w1b, b1b, 4.0, qpad1, 64)

    table2 = _table(x1, pos1p, w2a[:128], b2a, 256)
    edges2 = _sc_gather(table2, nbrf2)
    pos2p = jnp.pad(pos2, ((0, qpad2 - NS2), (0, 0)))
    qrep2 = jnp.broadcast_to(pos2p[:, None, :], (qpad2, K, 3)).reshape(qpad2 * K, 3)
    x2 = _conv(edges2, qrep2, w2a[128:], w2b, b2b, 16.0, qpad2, 128)

    return _tail(x2[:NS2], pos2, wga, bga, wgb, bgb, wfa, bfa, wfb, bfb)
